# baseline (speedup 1.0000x reference)
import functools

import jax
import jax.numpy as jnp
from jax import lax
from jax.experimental import pallas as pl
from jax.experimental.pallas import tpu as pltpu
from jax.experimental.pallas import tpu_sc as plsc

E = 16
K = 2
D = 2048
TOKENS = 16384
NW = 32
CHUNK = TOKENS // NW
LANES = 16
TBLK = 1024
CPB = TBLK // CHUNK


def _router_tc_body(x_ref, w_ref, b_ref, logits_ref, probs_ref, pt_ref):
    x = x_ref[...]
    w = w_ref[...]
    logits = lax.dot_general(x, w, (((1,), (1,)), ((), ())),
                             preferred_element_type=jnp.float32)
    logits = logits + b_ref[...]
    logits_ref[...] = logits
    m = jnp.max(logits, axis=-1, keepdims=True)
    ex = jnp.exp(logits - m)
    p = ex / jnp.sum(ex, axis=-1, keepdims=True)
    probs_ref[...] = p
    for j in range(CPB):
        pt_ref[pl.ds(j * E * CHUNK, E * CHUNK)] = (
            p[j * CHUNK:(j + 1) * CHUNK].T.reshape(E * CHUNK))


def _router_tc(x, w, b2d):
    nblk = TOKENS // TBLK
    return pl.pallas_call(
        _router_tc_body,
        grid=(nblk,),
        in_specs=[
            pl.BlockSpec((TBLK, D), lambda i: (i, 0)),
            pl.BlockSpec((E, D), lambda i: (0, 0)),
            pl.BlockSpec((1, E), lambda i: (0, 0)),
        ],
        out_specs=[
            pl.BlockSpec((TBLK, E), lambda i: (i, 0)),
            pl.BlockSpec((TBLK, E), lambda i: (i, 0)),
            pl.BlockSpec((CPB * E * CHUNK,), lambda i: (i,)),
        ],
        out_shape=[
            jax.ShapeDtypeStruct((TOKENS, E), jnp.float32),
            jax.ShapeDtypeStruct((TOKENS, E), jnp.float32),
            jax.ShapeDtypeStruct((NW * E * CHUNK,), jnp.float32),
        ],
        compiler_params=pltpu.CompilerParams(
            dimension_semantics=("parallel",)),
    )(x, w, b2d)


def _topk_sc_body(pt_hbm, w_hbm, i_hbm, pbuf, wbuf, ibuf):
    wid = lax.axis_index("s") * 2 + lax.axis_index("c")
    pltpu.sync_copy(pt_hbm.at[pl.ds(wid * (E * CHUNK), E * CHUNK)], pbuf)

    def group(g, _):
        base = g * LANES
        m1 = jnp.full((LANES,), -1.0, jnp.float32)
        m2 = jnp.full((LANES,), -1.0, jnp.float32)
        i1 = jnp.zeros((LANES,), jnp.int32)
        i2 = jnp.zeros((LANES,), jnp.int32)
        for e in range(E):
            p = pbuf[pl.ds(e * CHUNK + base, LANES)]
            gt1 = p > m1
            gt2 = p > m2
            i2 = jnp.where(gt1, i1, jnp.where(gt2, e, i2))
            m2 = jnp.where(gt1, m1, jnp.where(gt2, p, m2))
            i1 = jnp.where(gt1, e, i1)
            m1 = jnp.where(gt1, p, m1)
        rs = 1.0 / (m1 + m2)
        wbuf[pl.ds(base, LANES)] = m1 * rs
        wbuf[pl.ds(CHUNK + base, LANES)] = m2 * rs
        ibuf[pl.ds(base, LANES)] = i1
        ibuf[pl.ds(CHUNK + base, LANES)] = i2
        return 0

    lax.fori_loop(0, CHUNK // LANES, group, 0)
    for k in range(K):
        pltpu.sync_copy(wbuf.at[pl.ds(k * CHUNK, CHUNK)],
                        w_hbm.at[pl.ds(k * TOKENS + wid * CHUNK, CHUNK)])
        pltpu.sync_copy(ibuf.at[pl.ds(k * CHUNK, CHUNK)],
                        i_hbm.at[pl.ds(k * TOKENS + wid * CHUNK, CHUNK)])


@functools.cache
def _topk_sc():
    return pl.kernel(
        _topk_sc_body,
        mesh=plsc.VectorSubcoreMesh(core_axis_name="c", subcore_axis_name="s"),
        out_type=[
            jax.ShapeDtypeStruct((K * TOKENS,), jnp.float32),
            jax.ShapeDtypeStruct((K * TOKENS,), jnp.int32),
        ],
        scratch_types=[
            pltpu.VMEM((E * CHUNK,), jnp.float32),
            pltpu.VMEM((K * CHUNK,), jnp.float32),
            pltpu.VMEM((K * CHUNK,), jnp.int32),
        ],
    )


def kernel(inputs, W, b):
    B, S, _ = inputs.shape
    x = inputs.reshape(TOKENS, D)
    logits, probs, pt = _router_tc(x, W, b.reshape(1, E))
    wv, iv = _topk_sc()(pt)
    router_logits = logits.reshape(B, S, E)
    router_probs = probs.reshape(B, S, E)
    top_k_weights = jnp.moveaxis(wv.reshape(K, B, S), 0, -1)
    top_k_indices = jnp.moveaxis(iv.reshape(K, B, S), 0, -1)
    return (router_logits, router_probs, top_k_weights, top_k_indices)


# speedup vs baseline: 1.0204x; 1.0204x over previous
import functools

import jax
import jax.numpy as jnp
from jax import lax
from jax.experimental import pallas as pl
from jax.experimental.pallas import tpu as pltpu
from jax.experimental.pallas import tpu_sc as plsc

E = 16
K = 2
D = 2048
TOKENS = 16384
NW = 32
CHUNK = TOKENS // NW
LANES = 16
TBLK = 1024
CPB = TBLK // CHUNK


def _router_tc_body(x_ref, w_ref, b_ref, logits_ref, probs_ref, pt_ref):
    x = x_ref[...]
    w = w_ref[...]
    logits = lax.dot_general(x, w, (((1,), (1,)), ((), ())),
                             preferred_element_type=jnp.float32)
    logits = logits + b_ref[...]
    logits_ref[...] = logits
    m = jnp.max(logits, axis=-1, keepdims=True)
    ex = jnp.exp(logits - m)
    p = ex / jnp.sum(ex, axis=-1, keepdims=True)
    probs_ref[...] = p
    for j in range(CPB):
        pt_ref[pl.ds(j * E * CHUNK, E * CHUNK)] = (
            p[j * CHUNK:(j + 1) * CHUNK].T.reshape(E * CHUNK))


def _router_tc(x, w, b2d):
    nblk = TOKENS // TBLK
    return pl.pallas_call(
        _router_tc_body,
        grid=(nblk,),
        in_specs=[
            pl.BlockSpec((TBLK, D), lambda i: (i, 0)),
            pl.BlockSpec((E, D), lambda i: (0, 0)),
            pl.BlockSpec((1, E), lambda i: (0, 0)),
        ],
        out_specs=[
            pl.BlockSpec((TBLK, E), lambda i: (i, 0)),
            pl.BlockSpec((TBLK, E), lambda i: (i, 0)),
            pl.BlockSpec((CPB * E * CHUNK,), lambda i: (i,)),
        ],
        out_shape=[
            jax.ShapeDtypeStruct((TOKENS, E), jnp.float32),
            jax.ShapeDtypeStruct((TOKENS, E), jnp.float32),
            jax.ShapeDtypeStruct((NW * E * CHUNK,), jnp.float32),
        ],
        compiler_params=pltpu.CompilerParams(
            dimension_semantics=("parallel",)),
    )(x, w, b2d)


def _topk_sc_body(pt_hbm, o_hbm, pbuf, obuf):
    wid = lax.axis_index("s") * 2 + lax.axis_index("c")
    pltpu.sync_copy(pt_hbm.at[pl.ds(wid * (E * CHUNK), E * CHUNK)], pbuf)

    def group(g, _):
        base = g * LANES
        m1 = jnp.full((LANES,), -1.0, jnp.float32)
        m2 = jnp.full((LANES,), -1.0, jnp.float32)
        i1 = jnp.zeros((LANES,), jnp.int32)
        i2 = jnp.zeros((LANES,), jnp.int32)
        for e in range(E):
            p = pbuf[pl.ds(e * CHUNK + base, LANES)]
            gt1 = p > m1
            gt2 = p > m2
            i2 = jnp.where(gt1, i1, jnp.where(gt2, e, i2))
            m2 = jnp.where(gt1, m1, jnp.where(gt2, p, m2))
            i1 = jnp.where(gt1, e, i1)
            m1 = jnp.where(gt1, p, m1)
        rs = 1.0 / (m1 + m2)
        obuf[pl.ds(base, LANES)] = lax.bitcast_convert_type(m1 * rs, jnp.int32)
        obuf[pl.ds(CHUNK + base, LANES)] = lax.bitcast_convert_type(m2 * rs, jnp.int32)
        obuf[pl.ds(2 * CHUNK + base, LANES)] = i1
        obuf[pl.ds(3 * CHUNK + base, LANES)] = i2
        return 0

    lax.fori_loop(0, CHUNK // LANES, group, 0)
    for q in range(2 * K):
        pltpu.sync_copy(obuf.at[pl.ds(q * CHUNK, CHUNK)],
                        o_hbm.at[pl.ds(q * TOKENS + wid * CHUNK, CHUNK)])


@functools.cache
def _topk_sc():
    return pl.kernel(
        _topk_sc_body,
        mesh=plsc.VectorSubcoreMesh(core_axis_name="c", subcore_axis_name="s"),
        out_type=jax.ShapeDtypeStruct((2 * K * TOKENS,), jnp.int32),
        scratch_types=[
            pltpu.VMEM((E * CHUNK,), jnp.float32),
            pltpu.VMEM((2 * K * CHUNK,), jnp.int32),
        ],
    )


def kernel(inputs, W, b):
    B, S, _ = inputs.shape
    x = inputs.reshape(TOKENS, D)
    logits, probs, pt = _router_tc(x, W, b.reshape(1, E))
    out = _topk_sc()(pt)
    router_logits = logits.reshape(B, S, E)
    router_probs = probs.reshape(B, S, E)
    oi = jnp.moveaxis(out.reshape(2 * K, B, S), 0, -1)
    top_k_weights = lax.bitcast_convert_type(oi[..., :K], jnp.float32)
    top_k_indices = oi[..., K:]
    return (router_logits, router_probs, top_k_weights, top_k_indices)
